# Initial kernel scaffold; baseline (speedup 1.0000x reference)
import jax
import jax.numpy as jnp
from jax.experimental import pallas as pl


def kernel(x, edge_index, W, W_self):
    raise NotImplementedError("write your pallas kernel here")


# speedup vs baseline: 4.7799x; 4.7799x over previous
import functools

import jax
import jax.numpy as jnp
from jax import lax
from jax.experimental import pallas as pl
from jax.experimental.pallas import tpu as pltpu
from jax.experimental.pallas import tpu_sc as plsc

NC = 2
NS = 16
K = 128


@functools.lru_cache(maxsize=None)
def _make_sc_agg(N, D, R, E):
    assert R % NC == 0
    rel_per_core = R // NC
    e_per_tile = E // NS
    n_full = e_per_tile // K
    k_rem = e_per_tile - n_full * K
    rpt = (N // NS) // 8 * 8
    tail = N - NS * rpt
    assert tail % 8 == 0 and tail // 8 <= NS
    n_z128 = rpt // K
    z_rem = rpt - n_z128 * K

    mesh = plsc.VectorSubcoreMesh(core_axis_name="c", subcore_axis_name="s")

    scratch = [
        pltpu.VMEM((K,), jnp.int32),
        pltpu.VMEM((K,), jnp.int32),
        pltpu.VMEM((K, D), jnp.float32),
        pltpu.VMEM_SHARED((N, D), jnp.float32),
        pltpu.SemaphoreType.DMA,
    ]
    if k_rem:
        scratch += [
            pltpu.VMEM((k_rem,), jnp.int32),
            pltpu.VMEM((k_rem,), jnp.int32),
            pltpu.VMEM((k_rem, D), jnp.float32),
        ]

    @functools.partial(
        pl.kernel,
        mesh=mesh,
        out_type=jax.ShapeDtypeStruct((R, N, D), jnp.float32),
        scratch_types=scratch,
    )
    def sc_agg(x_hbm, ei_hbm, out_hbm, src_v, dst_v, rows_v, acc_sh, sem,
               *rem_bufs):
        c = lax.axis_index("c")
        s = lax.axis_index("s")
        ebase = s * e_per_tile
        row0 = s * rpt
        trow = NS * rpt + s * 8

        for phase in range(rel_per_core):
            r = c * rel_per_core + phase

            def _zrow(i, carry):
                for j in range(D // 16):
                    rows_v[i, pl.ds(j * 16, 16)] = jnp.zeros((16,), jnp.float32)
                return carry
            lax.fori_loop(0, K, _zrow, 0)
            for z in range(n_z128):
                pltpu.sync_copy(
                    rows_v,
                    acc_sh.at[pl.ds(row0 + z * K, K)])
            if z_rem:
                pltpu.sync_copy(
                    rows_v.at[pl.ds(0, z_rem)],
                    acc_sh.at[pl.ds(row0 + n_z128 * K, z_rem)])
            if tail:
                @pl.when(s < tail // 8)
                def _():
                    pltpu.sync_copy(rows_v.at[pl.ds(0, 8)],
                                    acc_sh.at[pl.ds(trow, 8)])
            plsc.subcore_barrier()

            src0 = (2 * r) * E + ebase
            dst0 = (2 * r + 1) * E + ebase

            def _chunk(j, carry):
                off = j * K
                pltpu.sync_copy(ei_hbm.at[pl.ds(src0 + off, K)], src_v)
                pltpu.sync_copy(ei_hbm.at[pl.ds(dst0 + off, K)], dst_v)
                pltpu.async_copy(x_hbm.at[src_v], rows_v, sem).wait()
                pltpu.sync_copy(rows_v, acc_sh.at[dst_v], add=True)
                return carry
            lax.fori_loop(0, n_full, _chunk, 0)
            if k_rem:
                srcr_v, dstr_v, rowsr_v = rem_bufs
                off = n_full * K
                pltpu.sync_copy(ei_hbm.at[pl.ds(src0 + off, k_rem)], srcr_v)
                pltpu.sync_copy(ei_hbm.at[pl.ds(dst0 + off, k_rem)], dstr_v)
                pltpu.async_copy(x_hbm.at[srcr_v], rowsr_v, sem).wait()
                pltpu.sync_copy(rowsr_v, acc_sh.at[dstr_v], add=True)
            plsc.subcore_barrier()

            pltpu.sync_copy(
                acc_sh.at[pl.ds(row0, rpt)],
                out_hbm.at[r, pl.ds(row0, rpt)])
            if tail:
                @pl.when(s < tail // 8)
                def _():
                    pltpu.sync_copy(acc_sh.at[pl.ds(trow, 8)],
                                    out_hbm.at[r, pl.ds(trow, 8)])

    return sc_agg


@functools.lru_cache(maxsize=None)
def _make_tc_mm(N, D, Dout, R, bm=1000):
    grid = N // bm

    def _mm_body(x_ref, a_ref, w_ref, ws_ref, o_ref):
        acc = jnp.dot(x_ref[...], ws_ref[...],
                      preferred_element_type=jnp.float32)
        for r in range(R):
            acc = acc + jnp.dot(a_ref[r], w_ref[r],
                                preferred_element_type=jnp.float32)
        o_ref[...] = acc

    return pl.pallas_call(
        _mm_body,
        grid=(grid,),
        in_specs=[
            pl.BlockSpec((bm, D), lambda i: (i, 0)),
            pl.BlockSpec((R, bm, D), lambda i: (0, i, 0)),
            pl.BlockSpec((R, D, Dout), lambda i: (0, 0, 0)),
            pl.BlockSpec((D, Dout), lambda i: (0, 0)),
        ],
        out_specs=pl.BlockSpec((bm, Dout), lambda i: (i, 0)),
        out_shape=jax.ShapeDtypeStruct((N, Dout), jnp.float32),
    )


def kernel(x, edge_index, W, W_self):
    B, N, D = x.shape
    R, _, E = edge_index.shape
    Dout = W_self.shape[1]
    x2 = x.reshape(N, D)
    agg = _make_sc_agg(N, D, R, E)(x2, edge_index.reshape(-1))
    out = _make_tc_mm(N, D, Dout, R)(x2, agg, W, W_self)
    return out.reshape(B, N, Dout)
